# baseline (speedup 1.0000x reference)
import functools

import numpy as np

import jax
import jax.numpy as jnp
from jax import lax
from jax.experimental import pallas as pl
from jax.experimental.pallas import tpu as pltpu
from jax.experimental.pallas import tpu_sc as plsc

_LANES = 16
_NBUF = 3

_PE_CACHE = {}


def _pos_encoding(seq_len, d_model):
    key = (seq_len, d_model)
    if key not in _PE_CACHE:
        pos = np.arange(seq_len, dtype=np.float32)[:, None]
        i = np.arange(0, d_model, 2, dtype=np.float32)[None, :]
        angle = pos / np.power(np.float32(10000.0), i / np.float32(d_model))
        pe = np.empty((seq_len, d_model), dtype=np.float32)
        pe[:, 0::2] = np.sin(angle)
        pe[:, 1::2] = np.cos(angle)
        u = pe.view(np.uint32)
        b = ((u + (((u >> 16) & 1) + 0x7FFF)) >> 16).astype(np.uint32)
        b = b.reshape(seq_len, d_model // 32, 2, 16)
        packed = (b[:, :, 1, :] << 16) | b[:, :, 0, :]
        packed = packed.reshape(seq_len, d_model // 2).astype(np.int32)
        _PE_CACHE[key] = jnp.asarray(packed)
    return _PE_CACHE[key]


@functools.partial(jax.jit, static_argnums=(3, 4, 5))
def _sc_embed(x, pe, table, batch, seq, d):
    info = plsc.get_sparse_core_info()
    nc, ns = info.num_cores, info.num_subcores
    nw = nc * ns
    s_per_w = seq // nw
    k = 16
    nchunk = s_per_w // k
    nsteps = nchunk * batch
    ngrp = d // (2 * _LANES)

    mesh = plsc.VectorSubcoreMesh(core_axis_name="c", subcore_axis_name="s")

    @functools.partial(
        pl.kernel,
        out_type=jax.ShapeDtypeStruct((batch, seq, d), jnp.float32),
        mesh=mesh,
        scratch_types=[
            pltpu.VMEM((_NBUF, k), jnp.int32),
            pltpu.VMEM((k, d // 2), jnp.int32),
            pltpu.VMEM((k, d), jnp.float32),
            pltpu.VMEM((k, d), jnp.float32),
            pltpu.VMEM((k, d), jnp.float32),
            pltpu.SemaphoreType.DMA,
            pltpu.SemaphoreType.DMA,
            pltpu.SemaphoreType.DMA,
            pltpu.SemaphoreType.DMA,
            pltpu.SemaphoreType.DMA,
            pltpu.SemaphoreType.DMA,
        ],
    )
    def run(x_hbm, pe_hbm, table_hbm, out_hbm,
            idx_v, pe_v, r0, r1, r2, g0, g1, g2, o0, o1, o2):
        rows = [r0, r1, r2]
        gsem = [g0, g1, g2]
        osem = [o0, o1, o2]
        wid = lax.axis_index("s") * nc + lax.axis_index("c")
        w0 = wid * s_per_w

        def cb_of(t):
            c, b = divmod(t, batch)
            return c, b

        def fire_gather(t):
            p = t % _NBUF
            c, b = cb_of(t)
            pltpu.sync_copy(x_hbm.at[b, pl.ds(w0 + c * k, k)], idx_v.at[p])
            pltpu.async_copy(table_hbm.at[idx_v.at[p]], rows[p], gsem[p])

        def wait_gather(t):
            p = t % _NBUF
            pltpu.make_async_copy(
                table_hbm.at[idx_v.at[p]], rows[p], gsem[p]).wait()

        def fire_store(t):
            p = t % _NBUF
            c, b = cb_of(t)
            pltpu.async_copy(
                rows[p], out_hbm.at[b, pl.ds(w0 + c * k, k)], osem[p])

        def wait_store(t):
            p = t % _NBUF
            c, b = cb_of(t)
            pltpu.make_async_copy(
                rows[p], out_hbm.at[b, pl.ds(w0 + c * k, k)], osem[p]).wait()

        def add_pe(p):
            buf = rows[p]

            def body(i, carry):
                r = i // ngrp
                g = i % ngrp
                v = pe_v[r, pl.ds(g * _LANES, _LANES)]
                shift = jnp.full((_LANES,), 16, jnp.int32)
                mask = jnp.full((_LANES,), -65536, jnp.int32)
                lo = lax.bitcast_convert_type(
                    lax.shift_left(v, shift), jnp.float32)
                hi = lax.bitcast_convert_type(
                    lax.bitwise_and(v, mask), jnp.float32)
                col = g * 2 * _LANES
                plsc.addupdate(buf.at[r, pl.ds(col, _LANES)], lo)
                plsc.addupdate(buf.at[r, pl.ds(col + _LANES, _LANES)], hi)
                return carry

            lax.fori_loop(0, k * ngrp, body, 0, unroll=4)

        pltpu.sync_copy(pe_hbm.at[pl.ds(w0, k)], pe_v)
        fire_gather(0)
        fire_gather(1)

        for t in range(nsteps):
            p = t % _NBUF
            wait_gather(t)
            if t % batch == 0 and t > 0:
                c = t // batch
                pltpu.sync_copy(pe_hbm.at[pl.ds(w0 + c * k, k)], pe_v)
            add_pe(p)
            fire_store(t)
            if t + 2 < nsteps:
                if t >= 1:
                    wait_store(t - 1)
                fire_gather(t + 2)

        for t in (nsteps - 3, nsteps - 2, nsteps - 1):
            wait_store(t)

    return run(x, pe, table)


def kernel(x, table):
    b, s = x.shape
    v, d = table.shape
    pe = _pos_encoding(s, d)
    return _sc_embed(x.astype(jnp.int32), pe, table, b, s, d)


# speedup vs baseline: 1.2731x; 1.2731x over previous
import functools

import numpy as np

import jax
import jax.numpy as jnp
from jax import lax
from jax.experimental import pallas as pl
from jax.experimental.pallas import tpu as pltpu
from jax.experimental.pallas import tpu_sc as plsc

_LANES = 16
_NBUF = 3

_PE_CACHE = {}


def _pos_encoding(seq_len, d_model):
    key = (seq_len, d_model)
    if key not in _PE_CACHE:
        pos = np.arange(seq_len, dtype=np.float32)[:, None]
        i = np.arange(0, d_model, 2, dtype=np.float32)[None, :]
        angle = pos / np.power(np.float32(10000.0), i / np.float32(d_model))
        pe = np.empty((seq_len, d_model), dtype=np.float32)
        pe[:, 0::2] = np.sin(angle)
        pe[:, 1::2] = np.cos(angle)
        _PE_CACHE[key] = jnp.asarray(pe)
    return _PE_CACHE[key]


@functools.partial(jax.jit, static_argnums=(3, 4, 5))
def _sc_embed(x, pe, table, batch, seq, d):
    info = plsc.get_sparse_core_info()
    nc, ns = info.num_cores, info.num_subcores
    nw = nc * ns
    s_per_w = seq // nw
    k = 16
    nchunk = s_per_w // k
    nsteps = nchunk * batch
    ncol = d // _LANES

    mesh = plsc.VectorSubcoreMesh(core_axis_name="c", subcore_axis_name="s")

    @functools.partial(
        pl.kernel,
        out_type=jax.ShapeDtypeStruct((batch, seq, d), jnp.float32),
        mesh=mesh,
        scratch_types=[
            pltpu.VMEM((batch, k), jnp.int32),
            pltpu.VMEM((k, d), jnp.float32),
            pltpu.VMEM((k, d), jnp.float32),
            pltpu.VMEM((k, d), jnp.float32),
            pltpu.VMEM((k, d), jnp.float32),
            pltpu.VMEM((k, d), jnp.float32),
            pltpu.SemaphoreType.DMA,
            pltpu.SemaphoreType.DMA,
            pltpu.SemaphoreType.DMA,
            pltpu.SemaphoreType.DMA,
            pltpu.SemaphoreType.DMA,
            pltpu.SemaphoreType.DMA,
            pltpu.SemaphoreType.DMA,
            pltpu.SemaphoreType.DMA,
        ],
    )
    def run(x_hbm, pe_hbm, table_hbm, out_hbm,
            idx_v, pe_v, r0, r1, r2, r3,
            g0, g1, g2, g3, o0, o1, o2, o3):
        rows = [r0, r1, r2, r3]
        gsem = [g0, g1, g2, g3]
        osem = [o0, o1, o2, o3]
        wid = lax.axis_index("s") * nc + lax.axis_index("c")
        w0 = wid * s_per_w

        def fire_gather(c, b):
            pltpu.sync_copy(x_hbm.at[b, pl.ds(w0 + c * k, k)], idx_v.at[b])
            pltpu.async_copy(table_hbm.at[idx_v.at[b]], rows[b], gsem[b])

        def wait_gather(b):
            pltpu.make_async_copy(
                table_hbm.at[idx_v.at[b]], rows[b], gsem[b]).wait()

        def fire_store(c, b):
            pltpu.async_copy(
                rows[b], out_hbm.at[b, pl.ds(w0 + c * k, k)], osem[b])

        def wait_store(c, b):
            pltpu.make_async_copy(
                rows[b], out_hbm.at[b, pl.ds(w0 + c * k, k)], osem[b]).wait()

        def add_pe(b):
            buf = rows[b]

            def body(i, carry):
                r = i // ncol
                col = (i % ncol) * _LANES
                plsc.addupdate(
                    buf.at[r, pl.ds(col, _LANES)],
                    pe_v[r, pl.ds(col, _LANES)],
                )
                return carry

            lax.fori_loop(0, k * ncol, body, 0, unroll=4)

        fire_gather(0, 0)
        fire_gather(0, 1)

        def chunk_body(c, carry):
            wait_gather(0)
            pltpu.sync_copy(pe_hbm.at[pl.ds(w0 + c * k, k)], pe_v)
            add_pe(0)
            fire_store(c, 0)

            @pl.when(c > 0)
            def _():
                wait_store(c - 1, 3)

            fire_gather(c, 2)
            wait_gather(1)
            add_pe(1)
            fire_store(c, 1)
            wait_store(c, 0)
            fire_gather(c, 3)
            wait_gather(2)
            add_pe(2)
            fire_store(c, 2)

            @pl.when(c + 1 < nchunk)
            def _():
                wait_store(c, 1)
                fire_gather(c + 1, 0)

            wait_gather(3)
            add_pe(3)
            fire_store(c, 3)

            @pl.when(c + 1 < nchunk)
            def _():
                wait_store(c, 2)
                fire_gather(c + 1, 1)

            return carry

        lax.fori_loop(0, nchunk, chunk_body, 0)

        for b in (1, 2, 3):
            wait_store(nchunk - 1, b)

    return run(x, pe, table)


def kernel(x, table):
    b, s = x.shape
    v, d = table.shape
    pe = _pos_encoding(s, d)
    return _sc_embed(x.astype(jnp.int32), pe, table, b, s, d)


# speedup vs baseline: 1.2740x; 1.0007x over previous
import functools

import numpy as np

import jax
import jax.numpy as jnp
from jax import lax
from jax.experimental import pallas as pl
from jax.experimental.pallas import tpu as pltpu
from jax.experimental.pallas import tpu_sc as plsc

_LANES = 16

_PE_CACHE = {}


def _pos_encoding(seq_len, d_model):
    key = (seq_len, d_model)
    if key not in _PE_CACHE:
        pos = np.arange(seq_len, dtype=np.float32)[:, None]
        i = np.arange(0, d_model, 2, dtype=np.float32)[None, :]
        angle = pos / np.power(np.float32(10000.0), i / np.float32(d_model))
        pe = np.empty((seq_len, d_model), dtype=np.float32)
        pe[:, 0::2] = np.sin(angle)
        pe[:, 1::2] = np.cos(angle)
        _PE_CACHE[key] = jnp.asarray(pe)
    return _PE_CACHE[key]


@functools.partial(jax.jit, static_argnums=(3, 4, 5))
def _sc_embed(x, pe, table, batch, seq, d):
    info = plsc.get_sparse_core_info()
    nc, ns = info.num_cores, info.num_subcores
    nw = nc * ns
    s_per_w = seq // nw
    k = 16
    nchunk = s_per_w // k
    ncol = d // _LANES

    mesh = plsc.VectorSubcoreMesh(core_axis_name="c", subcore_axis_name="s")

    @functools.partial(
        pl.kernel,
        out_type=jax.ShapeDtypeStruct((batch, seq, d), jnp.float32),
        mesh=mesh,
        scratch_types=[
            pltpu.VMEM((batch, k), jnp.int32),
            pltpu.VMEM((k, d), jnp.float32),
            pltpu.VMEM((k, d), jnp.float32),
            pltpu.VMEM((k, d), jnp.float32),
            pltpu.VMEM((k, d), jnp.float32),
            pltpu.VMEM((k, d), jnp.float32),
            pltpu.SemaphoreType.DMA,
            pltpu.SemaphoreType.DMA,
            pltpu.SemaphoreType.DMA,
            pltpu.SemaphoreType.DMA,
            pltpu.SemaphoreType.DMA,
            pltpu.SemaphoreType.DMA,
            pltpu.SemaphoreType.DMA,
            pltpu.SemaphoreType.DMA,
        ],
    )
    def run(x_hbm, pe_hbm, table_hbm, out_hbm,
            idx_v, pe_v, r0, r1, r2, r3,
            g0, g1, g2, g3, o0, o1, o2, o3):
        rows = [r0, r1, r2, r3]
        gsem = [g0, g1, g2, g3]
        osem = [o0, o1, o2, o3]
        wid = lax.axis_index("s") * nc + lax.axis_index("c")
        w0 = wid * s_per_w

        def fire_gather(c, b):
            pltpu.sync_copy(x_hbm.at[b, pl.ds(w0 + c * k, k)], idx_v.at[b])
            pltpu.async_copy(table_hbm.at[idx_v.at[b]], rows[b], gsem[b])

        def wait_gather(b):
            pltpu.make_async_copy(
                table_hbm.at[idx_v.at[b]], rows[b], gsem[b]).wait()

        def fire_store(c, b):
            pltpu.async_copy(
                rows[b], out_hbm.at[b, pl.ds(w0 + c * k, k)], osem[b])

        def wait_store(c, b):
            pltpu.make_async_copy(
                rows[b], out_hbm.at[b, pl.ds(w0 + c * k, k)], osem[b]).wait()

        def add_pe(b):
            buf = rows[b]

            def body(i, carry):
                r = i // ncol
                col = (i % ncol) * _LANES
                plsc.addupdate(
                    buf.at[r, pl.ds(col, _LANES)],
                    pe_v[r, pl.ds(col, _LANES)],
                )
                return carry

            lax.fori_loop(0, k * ncol, body, 0, unroll=4)

        fire_gather(0, 0)
        fire_gather(0, 1)

        def chunk_body(c, carry):
            wait_gather(0)
            pltpu.sync_copy(pe_hbm.at[pl.ds(w0 + c * k, k)], pe_v)
            add_pe(0)
            fire_store(c, 0)

            @pl.when(c > 0)
            def _():
                wait_store(c - 1, 3)

            fire_gather(c, 2)
            wait_gather(1)
            add_pe(1)
            fire_store(c, 1)
            wait_store(c, 0)
            fire_gather(c, 3)
            wait_gather(2)
            add_pe(2)
            fire_store(c, 2)

            @pl.when(c + 1 < nchunk)
            def _():
                wait_store(c, 1)
                fire_gather(c + 1, 0)

            wait_gather(3)
            add_pe(3)
            fire_store(c, 3)

            @pl.when(c + 1 < nchunk)
            def _():
                wait_store(c, 2)
                fire_gather(c + 1, 1)

            return carry

        lax.fori_loop(0, nchunk, chunk_body, 0)

        for b in (1, 2, 3):
            wait_store(nchunk - 1, b)

    return run(x, pe, table)


def kernel(x, table):
    b, s = x.shape
    v, d = table.shape
    pe = _pos_encoding(s, d)
    return _sc_embed(x.astype(jnp.int32), pe, table, b, s, d)
